# baseline (device time: 92099 ns/iter reference)
import jax
import jax.numpy as jnp
from jax import lax
from jax.experimental import pallas as pl
from jax.experimental.pallas import tpu as pltpu

N_DEV = 4
N_Q = 8
N_RIGHT = N_Q // 2

_CompilerParams = getattr(pltpu, "CompilerParams", None) or getattr(
    pltpu, "TPUCompilerParams"
)


def kernel(x, W1, W2):
    m, _ = x.shape
    d = W1.shape[1]
    n = W2.shape[1]
    chunk = m // N_DEV
    half = chunk // 2
    q_rows = chunk // N_Q

    def body(
        x_hbm, w1_ref, w2_ref, out_ref,
        h_ref, x_stage, w1b, w2b, rs_recv, ag_recv,
        stage_sems, rs_send_sems, rs_recv_sems, ag_send_sems, ag_recv_sems,
    ):
        my = lax.axis_index("i")
        left = (my - 1) % N_DEV
        right = (my + 1) % N_DEV

        starts = [
            ((my - 1) % N_DEV) * chunk,
            ((my + 1) % N_DEV) * chunk + half,
            ((my + 2) % N_DEV) * chunk,
            ((my + 2) % N_DEV) * chunk + half,
            ((my - 1) % N_DEV) * chunk + half,
            ((my + 1) % N_DEV) * chunk,
            my * chunk,
            my * chunk + half,
        ]

        def stage_dma(k):
            return pltpu.make_async_copy(
                x_hbm.at[pl.ds(starts[k], half), :],
                x_stage.at[k % 2],
                stage_sems.at[k % 2],
            )

        stage_dma(0).start()
        stage_dma(1).start()

        barrier_sem = pltpu.get_barrier_semaphore()
        for nbr in (left, right):
            pl.semaphore_signal(
                barrier_sem, inc=1,
                device_id=(nbr,), device_id_type=pl.DeviceIdType.MESH,
            )
        pl.semaphore_wait(barrier_sem, 2)

        w1b[...] = w1_ref[...].astype(jnp.bfloat16)

        def gemm1_half(k):
            stage_dma(k).wait()
            if k + 2 < len(starts):
                nxt = stage_dma(k + 2)
            h_ref[pl.ds(starts[k], half), :] = jnp.dot(
                x_stage[k % 2].astype(jnp.bfloat16), w1b[...],
                preferred_element_type=jnp.float32,
            ).astype(jnp.bfloat16)
            if k + 2 < len(starts):
                nxt.start()

        def row0(c, q):
            return c * chunk + q * q_rows

        def h_q(c, q):
            return h_ref.at[pl.ds(row0(c, q), q_rows), :]

        def rs_send_idx(q, s):
            return (my - 1 - s) % N_DEV if q < N_RIGHT else (my + 1 + s) % N_DEV

        def rs_recv_idx(q, s):
            return (my - 2 - s) % N_DEV if q < N_RIGHT else (my + 2 + s) % N_DEV

        def ag_hold_idx(q, t):
            return (my - t) % N_DEV if q < N_RIGHT else (my + t) % N_DEV

        def nbr_of(q):
            return right if q < N_RIGHT else left

        def start_rs(q, s):
            rdma = pltpu.make_async_remote_copy(
                src_ref=h_q(rs_send_idx(q, s), q),
                dst_ref=rs_recv.at[q, s],
                send_sem=rs_send_sems.at[q, s],
                recv_sem=rs_recv_sems.at[q, s],
                device_id=(nbr_of(q),),
                device_id_type=pl.DeviceIdType.MESH,
            )
            rdma.start()
            return rdma

        def start_ag(q, t):
            rdma = pltpu.make_async_remote_copy(
                src_ref=h_q(my, q) if t == 0 else ag_recv.at[q, t - 1],
                dst_ref=ag_recv.at[q, t],
                send_sem=ag_send_sems.at[q, t],
                recv_sem=ag_recv_sems.at[q, t],
                device_id=(nbr_of(q),),
                device_id_type=pl.DeviceIdType.MESH,
            )
            rdma.start()
            return rdma

        def gemm2_store(out_row, held):
            out_ref[pl.ds(out_row, q_rows), :] = jnp.dot(
                held, w2b[...], preferred_element_type=jnp.float32
            ).astype(jnp.bfloat16)

        CHAINS = tuple(
            q for pair in zip(range(N_RIGHT), range(N_RIGHT, N_Q)) for q in pair
        )

        rs = {}
        gemm1_half(0)
        for q in range(N_RIGHT):
            rs[q, 0] = start_rs(q, 0)
        gemm1_half(1)
        for q in range(N_RIGHT, N_Q):
            rs[q, 0] = start_rs(q, 0)
        w2b[...] = w2_ref[...].astype(jnp.bfloat16)
        for k in range(2, len(starts)):
            gemm1_half(k)

        ag = {}
        for s in range(N_DEV - 1):
            for q in CHAINS:
                rs[q, s].wait_recv()
                c = rs_recv_idx(q, s)
                h_ref[pl.ds(row0(c, q), q_rows), :] = (
                    rs_recv[q, s] + h_ref[pl.ds(row0(c, q), q_rows), :]
                )
                if s < N_DEV - 2:
                    rs[q, s + 1] = start_rs(q, s + 1)
                else:
                    ag[q, 0] = start_ag(q, 0)

        for q in CHAINS:
            gemm2_store(row0(my, q), h_ref[pl.ds(row0(my, q), q_rows), :])

        for t in range(N_DEV - 1):
            for q in CHAINS:
                ag[q, t].wait_recv()
                if t < N_DEV - 2:
                    ag[q, t + 1] = start_ag(q, t + 1)
                gemm2_store(row0(ag_hold_idx(q, t + 1), q), ag_recv[q, t])

        for rdma in list(rs.values()) + list(ag.values()):
            rdma.wait_send()

    return pl.pallas_call(
        body,
        out_shape=jax.ShapeDtypeStruct((m, n), jnp.bfloat16),
        in_specs=[
            pl.BlockSpec(memory_space=pl.ANY),
            pl.BlockSpec(memory_space=pltpu.VMEM),
            pl.BlockSpec(memory_space=pltpu.VMEM),
        ],
        out_specs=pl.BlockSpec(memory_space=pltpu.VMEM),
        scratch_shapes=[
            pltpu.VMEM((m, d), jnp.bfloat16),
            pltpu.VMEM((2, half, d), jnp.float32),
            pltpu.VMEM((d, d), jnp.bfloat16),
            pltpu.VMEM((d, n), jnp.bfloat16),
            pltpu.VMEM((N_Q, N_DEV - 1, q_rows, d), jnp.bfloat16),
            pltpu.VMEM((N_Q, N_DEV - 1, q_rows, d), jnp.bfloat16),
            pltpu.SemaphoreType.DMA((2,)),
            pltpu.SemaphoreType.DMA((N_Q, N_DEV - 1)),
            pltpu.SemaphoreType.DMA((N_Q, N_DEV - 1)),
            pltpu.SemaphoreType.DMA((N_Q, N_DEV - 1)),
            pltpu.SemaphoreType.DMA((N_Q, N_DEV - 1)),
        ],
        compiler_params=_CompilerParams(
            collective_id=0, vmem_limit_bytes=46 * 1024 * 1024
        ),
    )(x, W1, W2)


# device time: 91723 ns/iter; 1.0041x vs baseline; 1.0041x over previous
import jax
import jax.numpy as jnp
from jax import lax
from jax.experimental import pallas as pl
from jax.experimental.pallas import tpu as pltpu

N_DEV = 4
N_Q = 4

_CompilerParams = getattr(pltpu, "CompilerParams", None) or getattr(
    pltpu, "TPUCompilerParams"
)


def kernel(x, W1, W2):
    m, _ = x.shape
    d = W1.shape[1]
    n = W2.shape[1]
    chunk = m // N_DEV
    half = chunk // 2
    q_rows = chunk // N_Q

    def body(
        x_hbm, w1_ref, w2_ref, out_ref,
        h_ref, x_stage, w1b, w2b, rs_recv, ag_recv,
        stage_sems, rs_send_sems, rs_recv_sems, ag_send_sems, ag_recv_sems,
    ):
        my = lax.axis_index("i")
        left = (my - 1) % N_DEV
        right = (my + 1) % N_DEV

        starts = [
            ((my - 1) % N_DEV) * chunk,
            ((my + 1) % N_DEV) * chunk + half,
            ((my + 2) % N_DEV) * chunk,
            ((my + 2) % N_DEV) * chunk + half,
            ((my - 1) % N_DEV) * chunk + half,
            ((my + 1) % N_DEV) * chunk,
            my * chunk,
            my * chunk + half,
        ]

        def stage_dma(k):
            return pltpu.make_async_copy(
                x_hbm.at[pl.ds(starts[k], half), :],
                x_stage.at[k % 2],
                stage_sems.at[k % 2],
            )

        stage_dma(0).start()
        stage_dma(1).start()

        barrier_sem = pltpu.get_barrier_semaphore()
        for nbr in (left, right):
            pl.semaphore_signal(
                barrier_sem, inc=1,
                device_id=(nbr,), device_id_type=pl.DeviceIdType.MESH,
            )
        pl.semaphore_wait(barrier_sem, 2)

        w1b[...] = w1_ref[...].astype(jnp.bfloat16)

        def gemm1_half(k):
            stage_dma(k).wait()
            if k + 2 < len(starts):
                nxt = stage_dma(k + 2)
            h_ref[pl.ds(starts[k], half), :] = jnp.dot(
                x_stage[k % 2].astype(jnp.bfloat16), w1b[...],
                preferred_element_type=jnp.float32,
            ).astype(jnp.bfloat16)
            if k + 2 < len(starts):
                nxt.start()

        def row0(c, q):
            return c * chunk + q * q_rows

        def h_q(c, q):
            return h_ref.at[pl.ds(row0(c, q), q_rows), :]

        def rs_send_idx(q, s):
            return (my - 1 - s) % N_DEV if q < 2 else (my + 1 + s) % N_DEV

        def rs_recv_idx(q, s):
            return (my - 2 - s) % N_DEV if q < 2 else (my + 2 + s) % N_DEV

        def ag_hold_idx(q, t):
            return (my - t) % N_DEV if q < 2 else (my + t) % N_DEV

        def nbr_of(q):
            return right if q < 2 else left

        def start_rs(q, s):
            rdma = pltpu.make_async_remote_copy(
                src_ref=h_q(rs_send_idx(q, s), q),
                dst_ref=rs_recv.at[q, s],
                send_sem=rs_send_sems.at[q, s],
                recv_sem=rs_recv_sems.at[q, s],
                device_id=(nbr_of(q),),
                device_id_type=pl.DeviceIdType.MESH,
            )
            rdma.start()
            return rdma

        def start_ag(q, t):
            rdma = pltpu.make_async_remote_copy(
                src_ref=h_q(my, q) if t == 0 else ag_recv.at[q, t - 1],
                dst_ref=ag_recv.at[q, t],
                send_sem=ag_send_sems.at[q, t],
                recv_sem=ag_recv_sems.at[q, t],
                device_id=(nbr_of(q),),
                device_id_type=pl.DeviceIdType.MESH,
            )
            rdma.start()
            return rdma

        def gemm2_store(out_row, held):
            out_ref[pl.ds(out_row, q_rows), :] = jnp.dot(
                held, w2b[...], preferred_element_type=jnp.float32
            ).astype(jnp.bfloat16)

        CHAINS = (0, 2, 1, 3)

        rs = {}
        gemm1_half(0)
        for q in (0, 1):
            rs[q, 0] = start_rs(q, 0)
        gemm1_half(1)
        for q in (2, 3):
            rs[q, 0] = start_rs(q, 0)
        w2b[...] = w2_ref[...].astype(jnp.bfloat16)
        for k in range(2, len(starts)):
            gemm1_half(k)

        ag = {}
        for s in range(N_DEV - 1):
            for q in CHAINS:
                rs[q, s].wait_recv()
                c = rs_recv_idx(q, s)
                h_ref[pl.ds(row0(c, q), q_rows), :] = (
                    rs_recv[q, s] + h_ref[pl.ds(row0(c, q), q_rows), :]
                )
                if s < N_DEV - 2:
                    rs[q, s + 1] = start_rs(q, s + 1)
                else:
                    ag[q, 0] = start_ag(q, 0)

        for q in CHAINS:
            gemm2_store(row0(my, q), h_ref[pl.ds(row0(my, q), q_rows), :])

        for t in range(N_DEV - 1):
            for q in CHAINS:
                ag[q, t].wait_recv()
                if t < N_DEV - 2:
                    ag[q, t + 1] = start_ag(q, t + 1)
                gemm2_store(row0(ag_hold_idx(q, t + 1), q), ag_recv[q, t])

        for rdma in list(rs.values()) + list(ag.values()):
            rdma.wait_send()

    return pl.pallas_call(
        body,
        out_shape=jax.ShapeDtypeStruct((m, n), jnp.bfloat16),
        in_specs=[
            pl.BlockSpec(memory_space=pl.ANY),
            pl.BlockSpec(memory_space=pltpu.VMEM),
            pl.BlockSpec(memory_space=pltpu.VMEM),
        ],
        out_specs=pl.BlockSpec(memory_space=pltpu.VMEM),
        scratch_shapes=[
            pltpu.VMEM((m, d), jnp.bfloat16),
            pltpu.VMEM((2, half, d), jnp.float32),
            pltpu.VMEM((d, d), jnp.bfloat16),
            pltpu.VMEM((d, n), jnp.bfloat16),
            pltpu.VMEM((N_Q, N_DEV - 1, q_rows, d), jnp.bfloat16),
            pltpu.VMEM((N_Q, N_DEV - 1, q_rows, d), jnp.bfloat16),
            pltpu.SemaphoreType.DMA((2,)),
            pltpu.SemaphoreType.DMA((N_Q, N_DEV - 1)),
            pltpu.SemaphoreType.DMA((N_Q, N_DEV - 1)),
            pltpu.SemaphoreType.DMA((N_Q, N_DEV - 1)),
            pltpu.SemaphoreType.DMA((N_Q, N_DEV - 1)),
        ],
        compiler_params=_CompilerParams(
            collective_id=0, vmem_limit_bytes=46 * 1024 * 1024
        ),
    )(x, W1, W2)


# device time: 91710 ns/iter; 1.0042x vs baseline; 1.0001x over previous
import jax
import jax.numpy as jnp
from jax import lax
from jax.experimental import pallas as pl
from jax.experimental.pallas import tpu as pltpu

N_DEV = 4
N_Q = 4

_CompilerParams = getattr(pltpu, "CompilerParams", None) or getattr(
    pltpu, "TPUCompilerParams"
)


def kernel(x, W1, W2):
    m, _ = x.shape
    d = W1.shape[1]
    n = W2.shape[1]
    chunk = m // N_DEV
    half = chunk // 2
    q_rows = chunk // N_Q

    def body(
        x_hbm, w1_ref, w2_ref, out_ref,
        h_ref, x_stage, w1b, w2b, rs_recv, ag_recv,
        stage_sems, rs_send_sems, rs_recv_sems, ag_send_sems, ag_recv_sems,
    ):
        my = lax.axis_index("i")
        left = (my - 1) % N_DEV
        right = (my + 1) % N_DEV

        starts = [
            ((my - 1) % N_DEV) * chunk,
            ((my + 1) % N_DEV) * chunk + half,
            ((my + 2) % N_DEV) * chunk,
            ((my + 2) % N_DEV) * chunk + half,
            ((my - 1) % N_DEV) * chunk + half,
            ((my + 1) % N_DEV) * chunk,
            my * chunk,
            my * chunk + half,
        ]

        def stage_dma(k):
            return pltpu.make_async_copy(
                x_hbm.at[pl.ds(starts[k], half), :],
                x_stage.at[k % 2],
                stage_sems.at[k % 2],
            )

        stage_dma(0).start()
        stage_dma(1).start()

        barrier_sem = pltpu.get_barrier_semaphore()
        for nbr in (left, right):
            pl.semaphore_signal(
                barrier_sem, inc=1,
                device_id=(nbr,), device_id_type=pl.DeviceIdType.MESH,
            )

        w1b[...] = w1_ref[...].astype(jnp.bfloat16)

        def gemm1_half(k):
            stage_dma(k).wait()
            if k + 2 < len(starts):
                nxt = stage_dma(k + 2)
            h_ref[pl.ds(starts[k], half), :] = jnp.dot(
                x_stage[k % 2].astype(jnp.bfloat16), w1b[...],
                preferred_element_type=jnp.float32,
            ).astype(jnp.bfloat16)
            if k + 2 < len(starts):
                nxt.start()

        def row0(c, q):
            return c * chunk + q * q_rows

        def h_q(c, q):
            return h_ref.at[pl.ds(row0(c, q), q_rows), :]

        def rs_send_idx(q, s):
            return (my - 1 - s) % N_DEV if q < 2 else (my + 1 + s) % N_DEV

        def rs_recv_idx(q, s):
            return (my - 2 - s) % N_DEV if q < 2 else (my + 2 + s) % N_DEV

        def ag_hold_idx(q, t):
            return (my - t) % N_DEV if q < 2 else (my + t) % N_DEV

        def nbr_of(q):
            return right if q < 2 else left

        def start_rs(q, s):
            rdma = pltpu.make_async_remote_copy(
                src_ref=h_q(rs_send_idx(q, s), q),
                dst_ref=rs_recv.at[q, s],
                send_sem=rs_send_sems.at[q, s],
                recv_sem=rs_recv_sems.at[q, s],
                device_id=(nbr_of(q),),
                device_id_type=pl.DeviceIdType.MESH,
            )
            rdma.start()
            return rdma

        def start_ag(q, t):
            rdma = pltpu.make_async_remote_copy(
                src_ref=h_q(my, q) if t == 0 else ag_recv.at[q, t - 1],
                dst_ref=ag_recv.at[q, t],
                send_sem=ag_send_sems.at[q, t],
                recv_sem=ag_recv_sems.at[q, t],
                device_id=(nbr_of(q),),
                device_id_type=pl.DeviceIdType.MESH,
            )
            rdma.start()
            return rdma

        def gemm2_store(out_row, held):
            out_ref[pl.ds(out_row, q_rows), :] = jnp.dot(
                held, w2b[...], preferred_element_type=jnp.float32
            ).astype(jnp.bfloat16)

        CHAINS = (0, 2, 1, 3)

        rs = {}
        gemm1_half(0)
        pl.semaphore_wait(barrier_sem, 2)
        for q in (0, 1):
            rs[q, 0] = start_rs(q, 0)
        gemm1_half(1)
        for q in (2, 3):
            rs[q, 0] = start_rs(q, 0)
        w2b[...] = w2_ref[...].astype(jnp.bfloat16)
        for k in range(2, len(starts)):
            gemm1_half(k)

        ag = {}
        for s in range(N_DEV - 1):
            for q in CHAINS:
                rs[q, s].wait_recv()
                c = rs_recv_idx(q, s)
                h_ref[pl.ds(row0(c, q), q_rows), :] = (
                    rs_recv[q, s] + h_ref[pl.ds(row0(c, q), q_rows), :]
                )
                if s < N_DEV - 2:
                    rs[q, s + 1] = start_rs(q, s + 1)
                else:
                    ag[q, 0] = start_ag(q, 0)

        for q in CHAINS:
            gemm2_store(row0(my, q), h_ref[pl.ds(row0(my, q), q_rows), :])

        for t in range(N_DEV - 1):
            for q in CHAINS:
                ag[q, t].wait_recv()
                if t < N_DEV - 2:
                    ag[q, t + 1] = start_ag(q, t + 1)
                gemm2_store(row0(ag_hold_idx(q, t + 1), q), ag_recv[q, t])

        for rdma in list(rs.values()) + list(ag.values()):
            rdma.wait_send()

    return pl.pallas_call(
        body,
        out_shape=jax.ShapeDtypeStruct((m, n), jnp.bfloat16),
        in_specs=[
            pl.BlockSpec(memory_space=pl.ANY),
            pl.BlockSpec(memory_space=pltpu.VMEM),
            pl.BlockSpec(memory_space=pltpu.VMEM),
        ],
        out_specs=pl.BlockSpec(memory_space=pltpu.VMEM),
        scratch_shapes=[
            pltpu.VMEM((m, d), jnp.bfloat16),
            pltpu.VMEM((2, half, d), jnp.float32),
            pltpu.VMEM((d, d), jnp.bfloat16),
            pltpu.VMEM((d, n), jnp.bfloat16),
            pltpu.VMEM((N_Q, N_DEV - 1, q_rows, d), jnp.bfloat16),
            pltpu.VMEM((N_Q, N_DEV - 1, q_rows, d), jnp.bfloat16),
            pltpu.SemaphoreType.DMA((2,)),
            pltpu.SemaphoreType.DMA((N_Q, N_DEV - 1)),
            pltpu.SemaphoreType.DMA((N_Q, N_DEV - 1)),
            pltpu.SemaphoreType.DMA((N_Q, N_DEV - 1)),
            pltpu.SemaphoreType.DMA((N_Q, N_DEV - 1)),
        ],
        compiler_params=_CompilerParams(
            collective_id=0, vmem_limit_bytes=46 * 1024 * 1024
        ),
    )(x, W1, W2)


# device time: 88554 ns/iter; 1.0400x vs baseline; 1.0356x over previous
import jax
import jax.numpy as jnp
from jax import lax
from jax.experimental import pallas as pl
from jax.experimental.pallas import tpu as pltpu

N_DEV = 4
N_Q = 4

_CompilerParams = getattr(pltpu, "CompilerParams", None) or getattr(
    pltpu, "TPUCompilerParams"
)


def kernel(x, W1, W2):
    m, _ = x.shape
    d = W1.shape[1]
    n = W2.shape[1]
    chunk = m // N_DEV
    half = chunk // 2
    q_rows = chunk // N_Q

    def body(
        x_hbm, w1_ref, w2_ref, out_ref,
        h_ref, x_stage, w1b, w2b, rs_recv, ag_recv, out_stage,
        stage_sems, out_sems,
        rs_send_sems, rs_recv_sems, ag_send_sems, ag_recv_sems,
    ):
        my = lax.axis_index("i")
        left = (my - 1) % N_DEV
        right = (my + 1) % N_DEV

        starts = [
            ((my - 1) % N_DEV) * chunk,
            ((my + 1) % N_DEV) * chunk + half,
            ((my + 2) % N_DEV) * chunk,
            ((my + 2) % N_DEV) * chunk + half,
            ((my - 1) % N_DEV) * chunk + half,
            ((my + 1) % N_DEV) * chunk,
            my * chunk,
            my * chunk + half,
        ]

        def stage_dma(k):
            return pltpu.make_async_copy(
                x_hbm.at[pl.ds(starts[k], half), :],
                x_stage.at[k % 2],
                stage_sems.at[k % 2],
            )

        stage_dma(0).start()
        stage_dma(1).start()

        barrier_sem = pltpu.get_barrier_semaphore()
        for nbr in (left, right):
            pl.semaphore_signal(
                barrier_sem, inc=1,
                device_id=(nbr,), device_id_type=pl.DeviceIdType.MESH,
            )

        w1b[...] = w1_ref[...].astype(jnp.bfloat16)

        def gemm1_half(k):
            stage_dma(k).wait()
            if k + 2 < len(starts):
                nxt = stage_dma(k + 2)
            h_ref[pl.ds(starts[k], half), :] = jnp.dot(
                x_stage[k % 2].astype(jnp.bfloat16), w1b[...],
                preferred_element_type=jnp.float32,
            ).astype(jnp.bfloat16)
            if k + 2 < len(starts):
                nxt.start()

        def row0(c, q):
            return c * chunk + q * q_rows

        def h_q(c, q):
            return h_ref.at[pl.ds(row0(c, q), q_rows), :]

        def rs_send_idx(q, s):
            return (my - 1 - s) % N_DEV if q < 2 else (my + 1 + s) % N_DEV

        def rs_recv_idx(q, s):
            return (my - 2 - s) % N_DEV if q < 2 else (my + 2 + s) % N_DEV

        def ag_hold_idx(q, t):
            return (my - t) % N_DEV if q < 2 else (my + t) % N_DEV

        def nbr_of(q):
            return right if q < 2 else left

        def start_rs(q, s):
            rdma = pltpu.make_async_remote_copy(
                src_ref=h_q(rs_send_idx(q, s), q),
                dst_ref=rs_recv.at[q, s],
                send_sem=rs_send_sems.at[q, s],
                recv_sem=rs_recv_sems.at[q, s],
                device_id=(nbr_of(q),),
                device_id_type=pl.DeviceIdType.MESH,
            )
            rdma.start()
            return rdma

        def start_ag(q, t):
            rdma = pltpu.make_async_remote_copy(
                src_ref=h_q(my, q) if t == 0 else ag_recv.at[q, t - 1],
                dst_ref=ag_recv.at[q, t],
                send_sem=ag_send_sems.at[q, t],
                recv_sem=ag_recv_sems.at[q, t],
                device_id=(nbr_of(q),),
                device_id_type=pl.DeviceIdType.MESH,
            )
            rdma.start()
            return rdma

        out_n = [0]
        out_pending = {}

        def gemm2_store(out_row, held):
            slot = out_n[0] % 2
            out_n[0] += 1
            if slot in out_pending:
                out_pending[slot].wait()
            out_stage[slot] = jnp.dot(
                held, w2b[...], preferred_element_type=jnp.float32
            ).astype(jnp.bfloat16)
            dma = pltpu.make_async_copy(
                out_stage.at[slot],
                out_ref.at[pl.ds(out_row, q_rows), :],
                out_sems.at[slot],
            )
            dma.start()
            out_pending[slot] = dma

        CHAINS = (0, 2, 1, 3)

        rs = {}
        gemm1_half(0)
        pl.semaphore_wait(barrier_sem, 2)
        for q in (0, 1):
            rs[q, 0] = start_rs(q, 0)
        gemm1_half(1)
        for q in (2, 3):
            rs[q, 0] = start_rs(q, 0)
        w2b[...] = w2_ref[...].astype(jnp.bfloat16)
        for k in range(2, len(starts)):
            gemm1_half(k)

        ag = {}
        for s in range(N_DEV - 1):
            for q in CHAINS:
                rs[q, s].wait_recv()
                c = rs_recv_idx(q, s)
                h_ref[pl.ds(row0(c, q), q_rows), :] = (
                    rs_recv[q, s] + h_ref[pl.ds(row0(c, q), q_rows), :]
                )
                if s < N_DEV - 2:
                    rs[q, s + 1] = start_rs(q, s + 1)
                else:
                    ag[q, 0] = start_ag(q, 0)

        for q in CHAINS:
            gemm2_store(row0(my, q), h_ref[pl.ds(row0(my, q), q_rows), :])

        for t in range(N_DEV - 1):
            for q in CHAINS:
                ag[q, t].wait_recv()
                if t < N_DEV - 2:
                    ag[q, t + 1] = start_ag(q, t + 1)
                gemm2_store(row0(ag_hold_idx(q, t + 1), q), ag_recv[q, t])

        for dma in out_pending.values():
            dma.wait()
        for rdma in list(rs.values()) + list(ag.values()):
            rdma.wait_send()

    return pl.pallas_call(
        body,
        out_shape=jax.ShapeDtypeStruct((m, n), jnp.bfloat16),
        in_specs=[
            pl.BlockSpec(memory_space=pl.ANY),
            pl.BlockSpec(memory_space=pltpu.VMEM),
            pl.BlockSpec(memory_space=pltpu.VMEM),
        ],
        out_specs=pl.BlockSpec(memory_space=pl.ANY),
        scratch_shapes=[
            pltpu.VMEM((m, d), jnp.bfloat16),
            pltpu.VMEM((2, half, d), jnp.float32),
            pltpu.VMEM((d, d), jnp.bfloat16),
            pltpu.VMEM((d, n), jnp.bfloat16),
            pltpu.VMEM((N_Q, N_DEV - 1, q_rows, d), jnp.bfloat16),
            pltpu.VMEM((N_Q, N_DEV - 1, q_rows, d), jnp.bfloat16),
            pltpu.VMEM((2, q_rows, n), jnp.bfloat16),
            pltpu.SemaphoreType.DMA((2,)),
            pltpu.SemaphoreType.DMA((2,)),
            pltpu.SemaphoreType.DMA((N_Q, N_DEV - 1)),
            pltpu.SemaphoreType.DMA((N_Q, N_DEV - 1)),
            pltpu.SemaphoreType.DMA((N_Q, N_DEV - 1)),
            pltpu.SemaphoreType.DMA((N_Q, N_DEV - 1)),
        ],
        compiler_params=_CompilerParams(
            collective_id=0, vmem_limit_bytes=52 * 1024 * 1024
        ),
    )(x, W1, W2)
